# baseline (device time: 45056 ns/iter reference)
import jax
import jax.numpy as jnp
from jax import lax
from jax.experimental import pallas as pl
from jax.experimental.pallas import tpu as pltpu

N_DEV = 8
BFLY_MASKS = (4, 1, 3)
N_BLK = 4


def kernel(x, Win0, Wout0, Win1, Wout1, Win2, Wout2):
    b, d = x.shape
    _, hdim = Win0.shape
    out_rows = b // N_DEV
    blk_rows = b // N_BLK

    def body(x_ref, win0, wout0, win1, wout1, win2, wout2, out_ref,
             win_stage, wout_stage, acc_ref, flat_ref, bfly_ref, rs_ref,
             wsems, bfly_send, bfly_recv, rs_send, rs_recv):
        me = lax.axis_index("i")
        wins = [win0, win1, win2]
        wouts = [wout0, wout1, wout2]

        barrier_sem = pltpu.get_barrier_semaphore()
        for k in range(1, N_DEV):
            peer = lax.rem(me + k, N_DEV)
            pl.semaphore_signal(
                barrier_sem, inc=1,
                device_id=(peer,), device_id_type=pl.DeviceIdType.MESH,
            )
        pl.semaphore_wait(barrier_sem, N_DEV - 1)

        def start_layer_load(l):
            cw = pltpu.make_async_copy(wins[l], win_stage.at[l % 2],
                                       wsems.at[2 * l])
            co = pltpu.make_async_copy(wouts[l], wout_stage.at[l % 2],
                                       wsems.at[2 * l + 1])
            cw.start()
            co.start()
            return cw, co

        def bfly_rdma(l, s, blk):
            partner = me ^ BFLY_MASKS[s]
            return pltpu.make_async_remote_copy(
                src_ref=acc_ref.at[blk],
                dst_ref=bfly_ref.at[l, s, blk],
                send_sem=bfly_send.at[l, s, blk],
                recv_sem=bfly_recv.at[l, s, blk],
                device_id=(partner,),
                device_id_type=pl.DeviceIdType.MESH,
            )

        pending = start_layer_load(0)

        for l in range(3):
            pending[0].wait()
            pending[1].wait()
            if l < 2:
                pending = start_layer_load(l + 1)

            if l < 2:
                for blk in range(N_BLK):
                    if l == 0:
                        xin = x_ref[pl.ds(blk * blk_rows, blk_rows), :]
                    else:
                        xin = acc_ref[blk]
                    h = jnp.maximum(
                        jnp.dot(xin, win_stage[l % 2],
                                preferred_element_type=jnp.float32),
                        0.0,
                    )
                    acc_ref[blk] = jnp.dot(h, wout_stage[l % 2],
                                           preferred_element_type=jnp.float32)
                    bfly_rdma(l, 0, blk).start()
                for s in range(len(BFLY_MASKS)):
                    for blk in range(N_BLK):
                        bfly_rdma(l, s, blk).wait()
                        acc_ref[blk] = acc_ref[blk] + bfly_ref[l, s, blk]
                        if s + 1 < len(BFLY_MASKS):
                            bfly_rdma(l, s + 1, blk).start()
            else:
                rdmas = {}
                for k in range(1, N_DEV):
                    peer = lax.rem(me + k, N_DEV)
                    rdmas[k] = pltpu.make_async_remote_copy(
                        src_ref=flat_ref.at[pl.ds(peer * out_rows, out_rows)],
                        dst_ref=rs_ref.at[k - 1],
                        send_sem=rs_send.at[k - 1],
                        recv_sem=rs_recv.at[k - 1],
                        device_id=(peer,),
                        device_id_type=pl.DeviceIdType.MESH,
                    )
                peers_per_blk = N_DEV // N_BLK
                for blk in range(N_BLK):
                    xin = acc_ref[blk]
                    h = jnp.maximum(
                        jnp.dot(xin, win_stage[l % 2],
                                preferred_element_type=jnp.float32),
                        0.0,
                    )
                    flat_ref[pl.ds(blk * blk_rows, blk_rows), :] = jnp.dot(
                        h, wout_stage[l % 2],
                        preferred_element_type=jnp.float32)
                    for k in range(1, N_DEV):
                        peer = lax.rem(me + k, N_DEV)
                        in_blk = peer // peers_per_blk == blk

                        @pl.when(in_blk)
                        def _(k=k):
                            rdmas[k].start()

                for k in range(1, N_DEV):
                    rdmas[k].wait_send()
                for k in range(1, N_DEV):
                    rdmas[k].wait_recv()
                out = flat_ref[pl.ds(me * out_rows, out_rows), :]
                for k in range(N_DEV - 1):
                    out = out + rs_ref[k]
                out_ref[...] = out

    return pl.pallas_call(
        body,
        out_shape=jax.ShapeDtypeStruct((out_rows, d), jnp.float32),
        in_specs=[pl.BlockSpec(memory_space=pltpu.VMEM)]
        + [pl.BlockSpec(memory_space=pl.ANY)] * 6,
        out_specs=pl.BlockSpec(memory_space=pltpu.VMEM),
        scratch_shapes=[
            pltpu.VMEM((2, d, hdim), jnp.float32),
            pltpu.VMEM((2, hdim, d), jnp.float32),
            pltpu.VMEM((N_BLK, blk_rows, d), jnp.float32),
            pltpu.VMEM((b, d), jnp.float32),
            pltpu.VMEM((2, len(BFLY_MASKS), N_BLK, blk_rows, d),
                       jnp.float32),
            pltpu.VMEM((N_DEV - 1, out_rows, d), jnp.float32),
            pltpu.SemaphoreType.DMA((6,)),
            pltpu.SemaphoreType.DMA((2, len(BFLY_MASKS), N_BLK)),
            pltpu.SemaphoreType.DMA((2, len(BFLY_MASKS), N_BLK)),
            pltpu.SemaphoreType.DMA((N_DEV - 1,)),
            pltpu.SemaphoreType.DMA((N_DEV - 1,)),
        ],
        compiler_params=pltpu.CompilerParams(
            collective_id=0,
            vmem_limit_bytes=60 * 1024 * 1024,
        ),
    )(x, Win0, Wout0, Win1, Wout1, Win2, Wout2)


# device time: 41946 ns/iter; 1.0741x vs baseline; 1.0741x over previous
import jax
import jax.numpy as jnp
from jax import lax
from jax.experimental import pallas as pl
from jax.experimental.pallas import tpu as pltpu

N_DEV = 8
BFLY_MASKS = (4, 1, 3)
N_BLK = 2


def kernel(x, Win0, Wout0, Win1, Wout1, Win2, Wout2):
    b, d = x.shape
    _, hdim = Win0.shape
    out_rows = b // N_DEV
    blk_rows = b // N_BLK

    def body(x_ref, win0, wout0, win1, wout1, win2, wout2, out_ref,
             win_stage, wout_stage, acc_ref, flat_ref, bfly_ref, rs_ref,
             wsems, bfly_send, bfly_recv, rs_send, rs_recv):
        me = lax.axis_index("i")
        wins = [win0, win1, win2]
        wouts = [wout0, wout1, wout2]

        def start_layer_load(l):
            cw = pltpu.make_async_copy(wins[l], win_stage.at[l % 2],
                                       wsems.at[2 * l])
            co = pltpu.make_async_copy(wouts[l], wout_stage.at[l % 2],
                                       wsems.at[2 * l + 1])
            cw.start()
            co.start()
            return cw, co

        def bfly_rdma(l, s, blk):
            partner = me ^ BFLY_MASKS[s]
            return pltpu.make_async_remote_copy(
                src_ref=acc_ref.at[blk],
                dst_ref=bfly_ref.at[l, s, blk],
                send_sem=bfly_send.at[l, s, blk],
                recv_sem=bfly_recv.at[l, s, blk],
                device_id=(partner,),
                device_id_type=pl.DeviceIdType.MESH,
            )

        pending = start_layer_load(0)

        barrier_sem = pltpu.get_barrier_semaphore()
        for k in range(1, N_DEV):
            peer = lax.rem(me + k, N_DEV)
            pl.semaphore_signal(
                barrier_sem, inc=1,
                device_id=(peer,), device_id_type=pl.DeviceIdType.MESH,
            )
        pl.semaphore_wait(barrier_sem, N_DEV - 1)

        for l in range(3):
            pending[0].wait()
            pending[1].wait()
            if l < 2:
                pending = start_layer_load(l + 1)

            if l < 2:
                for blk in range(N_BLK):
                    if l == 0:
                        xin = x_ref[pl.ds(blk * blk_rows, blk_rows), :]
                    else:
                        xin = acc_ref[blk]
                    h = jnp.maximum(
                        jnp.dot(xin, win_stage[l % 2],
                                preferred_element_type=jnp.float32),
                        0.0,
                    )
                    acc_ref[blk] = jnp.dot(h, wout_stage[l % 2],
                                           preferred_element_type=jnp.float32)
                    bfly_rdma(l, 0, blk).start()
                for s in range(len(BFLY_MASKS)):
                    for blk in range(N_BLK):
                        bfly_rdma(l, s, blk).wait()
                        acc_ref[blk] = acc_ref[blk] + bfly_ref[l, s, blk]
                        if s + 1 < len(BFLY_MASKS):
                            bfly_rdma(l, s + 1, blk).start()
            else:
                rdmas = {}
                for k in range(1, N_DEV):
                    peer = lax.rem(me + k, N_DEV)
                    rdmas[k] = pltpu.make_async_remote_copy(
                        src_ref=flat_ref.at[pl.ds(peer * out_rows, out_rows)],
                        dst_ref=rs_ref.at[k - 1],
                        send_sem=rs_send.at[k - 1],
                        recv_sem=rs_recv.at[k - 1],
                        device_id=(peer,),
                        device_id_type=pl.DeviceIdType.MESH,
                    )
                peers_per_blk = N_DEV // N_BLK
                for blk in range(N_BLK):
                    xin = acc_ref[blk]
                    h = jnp.maximum(
                        jnp.dot(xin, win_stage[l % 2],
                                preferred_element_type=jnp.float32),
                        0.0,
                    )
                    flat_ref[pl.ds(blk * blk_rows, blk_rows), :] = jnp.dot(
                        h, wout_stage[l % 2],
                        preferred_element_type=jnp.float32)
                    for k in range(1, N_DEV):
                        peer = lax.rem(me + k, N_DEV)
                        in_blk = peer // peers_per_blk == blk

                        @pl.when(in_blk)
                        def _(k=k):
                            rdmas[k].start()

                for k in range(1, N_DEV):
                    rdmas[k].wait_send()
                for k in range(1, N_DEV):
                    rdmas[k].wait_recv()
                out = flat_ref[pl.ds(me * out_rows, out_rows), :]
                for k in range(N_DEV - 1):
                    out = out + rs_ref[k]
                out_ref[...] = out

    return pl.pallas_call(
        body,
        out_shape=jax.ShapeDtypeStruct((out_rows, d), jnp.float32),
        in_specs=[pl.BlockSpec(memory_space=pltpu.VMEM)]
        + [pl.BlockSpec(memory_space=pl.ANY)] * 6,
        out_specs=pl.BlockSpec(memory_space=pltpu.VMEM),
        scratch_shapes=[
            pltpu.VMEM((2, d, hdim), jnp.float32),
            pltpu.VMEM((2, hdim, d), jnp.float32),
            pltpu.VMEM((N_BLK, blk_rows, d), jnp.float32),
            pltpu.VMEM((b, d), jnp.float32),
            pltpu.VMEM((2, len(BFLY_MASKS), N_BLK, blk_rows, d),
                       jnp.float32),
            pltpu.VMEM((N_DEV - 1, out_rows, d), jnp.float32),
            pltpu.SemaphoreType.DMA((6,)),
            pltpu.SemaphoreType.DMA((2, len(BFLY_MASKS), N_BLK)),
            pltpu.SemaphoreType.DMA((2, len(BFLY_MASKS), N_BLK)),
            pltpu.SemaphoreType.DMA((N_DEV - 1,)),
            pltpu.SemaphoreType.DMA((N_DEV - 1,)),
        ],
        compiler_params=pltpu.CompilerParams(
            collective_id=0,
            vmem_limit_bytes=60 * 1024 * 1024,
        ),
    )(x, Win0, Wout0, Win1, Wout1, Win2, Wout2)
